# Initial kernel scaffold; baseline (speedup 1.0000x reference)
import jax
import jax.numpy as jnp
from jax.experimental import pallas as pl


def kernel(z1, z2, negative_samples):
    raise NotImplementedError("write your pallas kernel here")


# speedup vs baseline: 1.5025x; 1.5025x over previous
import functools

import jax
import jax.numpy as jnp
from jax import lax
from jax.experimental import pallas as pl
from jax.experimental.pallas import tpu as pltpu
from jax.experimental.pallas import tpu_sc as plsc

N = 10000
D = 128
NEG = 32

NW = 32
ROWS_PER_W = 320
NPAD = NW * ROWS_PER_W
GROUP = 4
N_GROUPS = ROWS_PER_W // GROUP
BLK = 16
GROUPS_PER_BLK = BLK // GROUP
N_BLKS = ROWS_PER_W // BLK


def _tc_normalize_body(z1_ref, z2_ref, o1_ref, o2_ref):
    x1 = z1_ref[...]
    x2 = z2_ref[...]
    n1 = jnp.maximum(jnp.sqrt(jnp.sum(x1 * x1, axis=1, keepdims=True)), 1e-12)
    n2 = jnp.maximum(jnp.sqrt(jnp.sum(x2 * x2, axis=1, keepdims=True)), 1e-12)
    o1_ref[...] = x1 / n1
    o2_ref[...] = x2 / n2


def _tc_normalize(z1p, z2p):
    blk = 256
    grid = (NPAD // blk,)
    spec = pl.BlockSpec((blk, D), lambda i: (i, 0))
    return pl.pallas_call(
        _tc_normalize_body,
        grid=grid,
        in_specs=[spec, spec],
        out_specs=[spec, spec],
        out_shape=[
            jax.ShapeDtypeStruct((NPAD, D), jnp.float32),
            jax.ShapeDtypeStruct((NPAD, D), jnp.float32),
        ],
    )(z1p, z2p)


def _sc_score_body(z1n_hbm, z2n_hbm, neg_hbm, out_hbm,
                   idx_v, z1_v, z2_v, gbuf, out_v, sem):
    info = plsc.get_sparse_core_info()
    nc = info.num_cores
    wid = lax.axis_index("s") * nc + lax.axis_index("c")
    base = wid * ROWS_PER_W

    pltpu.sync_copy(neg_hbm.at[pl.ds(wid * N_GROUPS, N_GROUPS)], idx_v)
    pltpu.sync_copy(z1n_hbm.at[pl.ds(base, ROWS_PER_W)], z1_v)
    pltpu.sync_copy(z2n_hbm.at[pl.ds(base, ROWS_PER_W)], z2_v)

    lanes = jnp.arange(16, dtype=jnp.int32)
    inv_neg = 1.0 / NEG

    dnums = lax.GatherDimensionNumbers(
        offset_dims=(), collapsed_slice_dims=(0,), start_index_map=(0,))

    def lane_sum(v):
        for sh in (8, 4, 2, 1):
            perm = (lanes ^ sh)[:, None]
            v = v + lax.gather(
                v, perm, dimension_numbers=dnums, slice_sizes=(1,),
                mode=lax.GatherScatterMode.PROMISE_IN_BOUNDS)
        return v

    def blk_body(blk, carry):
        res = jnp.zeros((16,), jnp.float32)
        for gsub in range(GROUPS_PER_BLK):
            g = blk * GROUPS_PER_BLK + gsub
            pltpu.async_copy(z2n_hbm.at[idx_v.at[g]], gbuf, sem).wait()
            for r in range(GROUP):
                row = blk * BLK + gsub * GROUP + r
                def j_body(j, acc):
                    return tuple(
                        acc[d] + gbuf[r * NEG + j, pl.ds(d * 16, 16)]
                        for d in range(8)
                    )
                acc = lax.fori_loop(
                    0, NEG, j_body,
                    tuple(jnp.zeros((16,), jnp.float32) for _ in range(8)))
                dotv = jnp.zeros((16,), jnp.float32)
                for d in range(8):
                    diff = z2_v[row, pl.ds(d * 16, 16)] - acc[d] * inv_neg
                    dotv = dotv + z1_v[row, pl.ds(d * 16, 16)] * diff
                s = lane_sum(dotv)
                res = jnp.where(lanes == gsub * GROUP + r, s, res)
        out_v[pl.ds(blk * BLK, BLK)] = res
        return carry

    lax.fori_loop(0, N_BLKS, blk_body, 0)
    pltpu.sync_copy(out_v, out_hbm.at[pl.ds(base, ROWS_PER_W)])


def _sc_score(z1n, z2n, neg_r):
    mesh = plsc.VectorSubcoreMesh(core_axis_name="c", subcore_axis_name="s")
    kfn = functools.partial(
        pl.kernel,
        mesh=mesh,
        out_type=jax.ShapeDtypeStruct((NPAD,), jnp.float32),
        scratch_types=[
            pltpu.VMEM((NW * N_GROUPS // NW, 128), jnp.int32),
            pltpu.VMEM((ROWS_PER_W, D), jnp.float32),
            pltpu.VMEM((ROWS_PER_W, D), jnp.float32),
            pltpu.VMEM((GROUP * NEG, D), jnp.float32),
            pltpu.VMEM((ROWS_PER_W,), jnp.float32),
            pltpu.SemaphoreType.DMA,
        ],
    )(_sc_score_body)
    return kfn(z1n, z2n, neg_r)


def kernel(z1, z2, negative_samples):
    z1p = jnp.pad(z1, ((0, NPAD - N), (0, 0)))
    z2p = jnp.pad(z2, ((0, NPAD - N), (0, 0)))
    neg = jnp.pad(negative_samples.astype(jnp.int32), ((0, NPAD - N), (0, 0)))
    neg_r = neg.reshape(NPAD * NEG // 128, 128)
    z1n, z2n = _tc_normalize(z1p, z2p)
    out = _sc_score(z1n, z2n, neg_r)
    return out[:N]


# speedup vs baseline: 1.6777x; 1.1166x over previous
import functools

import jax
import jax.numpy as jnp
from jax import lax
from jax.experimental import pallas as pl
from jax.experimental.pallas import tpu as pltpu
from jax.experimental.pallas import tpu_sc as plsc

N = 10000
D = 128
NEG = 32

NW = 32
ROWS_PER_W = 320
NPAD = NW * ROWS_PER_W
GROUP = 4
N_GROUPS = ROWS_PER_W // GROUP
BLK = 16
GROUPS_PER_BLK = BLK // GROUP
N_BLKS = ROWS_PER_W // BLK


def _tc_normalize_body(z1_ref, z2_ref, o1_ref, o2_ref):
    x1 = z1_ref[...]
    x2 = z2_ref[...]
    n1 = jnp.maximum(jnp.sqrt(jnp.sum(x1 * x1, axis=1, keepdims=True)), 1e-12)
    n2 = jnp.maximum(jnp.sqrt(jnp.sum(x2 * x2, axis=1, keepdims=True)), 1e-12)
    o1_ref[...] = x1 / n1
    o2_ref[...] = x2 / n2


def _tc_normalize(z1p, z2p):
    blk = 256
    grid = (NPAD // blk,)
    spec = pl.BlockSpec((blk, D), lambda i: (i, 0))
    return pl.pallas_call(
        _tc_normalize_body,
        grid=grid,
        in_specs=[spec, spec],
        out_specs=[spec, spec],
        out_shape=[
            jax.ShapeDtypeStruct((NPAD, D), jnp.float32),
            jax.ShapeDtypeStruct((NPAD, D), jnp.float32),
        ],
    )(z1p, z2p)


def _sc_score_body(z1n_hbm, z2n_hbm, neg_hbm, out_hbm,
                   idx_v, z1_v, z2_v, gbuf0, gbuf1, out_v,
                   sem0, sem1, semz):
    info = plsc.get_sparse_core_info()
    nc = info.num_cores
    wid = lax.axis_index("s") * nc + lax.axis_index("c")
    base = wid * ROWS_PER_W

    pltpu.sync_copy(neg_hbm.at[pl.ds(wid * N_GROUPS, N_GROUPS)], idx_v)
    z1_cp = pltpu.make_async_copy(
        z1n_hbm.at[pl.ds(base, ROWS_PER_W)], z1_v, semz)
    z2_cp = pltpu.make_async_copy(
        z2n_hbm.at[pl.ds(base, ROWS_PER_W)], z2_v, semz)
    z1_cp.start()
    z2_cp.start()

    bufs = (gbuf0, gbuf1)
    sems = (sem0, sem1)

    def gather_start(g, b):
        pltpu.make_async_copy(
            z2n_hbm.at[idx_v.at[g]], bufs[b], sems[b]).start()

    def gather_wait(g, b):
        pltpu.make_async_copy(
            z2n_hbm.at[idx_v.at[g]], bufs[b], sems[b]).wait()

    gather_start(0, 0)
    gather_start(1, 1)
    z1_cp.wait()
    z2_cp.wait()

    lanes = jnp.arange(16, dtype=jnp.int32)
    inv_neg = 1.0 / NEG

    dnums = lax.GatherDimensionNumbers(
        offset_dims=(), collapsed_slice_dims=(0,), start_index_map=(0,))

    def lane_sum(v):
        for sh in (8, 4, 2, 1):
            perm = (lanes ^ sh)[:, None]
            v = v + lax.gather(
                v, perm, dimension_numbers=dnums, slice_sizes=(1,),
                mode=lax.GatherScatterMode.PROMISE_IN_BOUNDS)
        return v

    def blk_body(blk, carry):
        res = jnp.zeros((16,), jnp.float32)
        for gsub in range(GROUPS_PER_BLK):
            g = blk * GROUPS_PER_BLK + gsub
            b = gsub % 2
            gather_wait(g, b)
            gb = bufs[b]
            for r in range(GROUP):
                row = blk * BLK + gsub * GROUP + r
                z1r = [z1_v[row, pl.ds(d * 16, 16)] for d in range(8)]

                def jc_body(jc, dotacc):
                    jb = jc * 8
                    for jj in range(8):
                        for d in range(8):
                            dotacc = dotacc + z1r[d] * gb[
                                r * NEG + jb + jj, pl.ds(d * 16, 16)]
                    return dotacc
                negacc = lax.fori_loop(
                    0, NEG // 8, jc_body, jnp.zeros((16,), jnp.float32))

                posacc = jnp.zeros((16,), jnp.float32)
                for d in range(8):
                    posacc = posacc + z1r[d] * z2_v[row, pl.ds(d * 16, 16)]

                s = lane_sum(posacc - inv_neg * negacc)
                res = jnp.where(lanes == gsub * GROUP + r, s, res)
            @pl.when(g + 2 < N_GROUPS)
            def _():
                gather_start(g + 2, b)
        out_v[pl.ds(blk * BLK, BLK)] = res
        return carry

    lax.fori_loop(0, N_BLKS, blk_body, 0)
    pltpu.sync_copy(out_v, out_hbm.at[pl.ds(base, ROWS_PER_W)])


def _sc_score(z1n, z2n, neg_r):
    mesh = plsc.VectorSubcoreMesh(core_axis_name="c", subcore_axis_name="s")
    kfn = functools.partial(
        pl.kernel,
        mesh=mesh,
        out_type=jax.ShapeDtypeStruct((NPAD,), jnp.float32),
        scratch_types=[
            pltpu.VMEM((NW * N_GROUPS // NW, 128), jnp.int32),
            pltpu.VMEM((ROWS_PER_W, D), jnp.float32),
            pltpu.VMEM((ROWS_PER_W, D), jnp.float32),
            pltpu.VMEM((GROUP * NEG, D), jnp.float32),
            pltpu.VMEM((GROUP * NEG, D), jnp.float32),
            pltpu.VMEM((ROWS_PER_W,), jnp.float32),
            pltpu.SemaphoreType.DMA,
            pltpu.SemaphoreType.DMA,
            pltpu.SemaphoreType.DMA,
        ],
    )(_sc_score_body)
    return kfn(z1n, z2n, neg_r)


def kernel(z1, z2, negative_samples):
    z1p = jnp.pad(z1, ((0, NPAD - N), (0, 0)))
    z2p = jnp.pad(z2, ((0, NPAD - N), (0, 0)))
    neg = jnp.pad(negative_samples.astype(jnp.int32), ((0, NPAD - N), (0, 0)))
    neg_r = neg.reshape(NPAD * NEG // 128, 128)
    z1n, z2n = _tc_normalize(z1p, z2p)
    out = _sc_score(z1n, z2n, neg_r)
    return out[:N]
